# baseline (speedup 1.0000x reference)
import jax
import jax.numpy as jnp
from jax.experimental import pallas as pl
from jax.experimental.pallas import tpu as pltpu

NUM_TOKENS = 32768
MODEL_DIM = 1024
NUM_EXPERTS = 64
BLOCK_T = 2048
NUM_BLOCKS = NUM_TOKENS // BLOCK_T


def _gate_body(x_ref, w_ref, idx_ref, loc_ref, gate_ref, laux_ref,
               me_acc, cnt_acc):
    i = pl.program_id(0)

    @pl.when(i == 0)
    def _init():
        me_acc[...] = jnp.zeros_like(me_acc)
        cnt_acc[...] = jnp.zeros_like(cnt_acc)

    x = x_ref[...]
    w = w_ref[...]
    logits = jax.lax.dot_general(
        x, w, (((1,), (1,)), ((), ())),
        preferred_element_type=jnp.float32)

    rowmax = jnp.max(logits, axis=1, keepdims=True)
    eidx_f = jax.lax.broadcasted_iota(
        jnp.int32, (BLOCK_T, NUM_EXPERTS), 1).astype(jnp.float32)
    is_max = logits == rowmax
    idx_f = jnp.min(jnp.where(is_max, eidx_f, float(NUM_EXPERTS)),
                    axis=1, keepdims=True)

    exps = jnp.exp(logits - rowmax)
    mask = (eidx_f == idx_f).astype(jnp.float32)

    r = jax.lax.broadcasted_iota(jnp.int32, (BLOCK_T, BLOCK_T), 0)
    c = jax.lax.broadcasted_iota(jnp.int32, (BLOCK_T, BLOCK_T), 1)
    ltri = (c < r).astype(jnp.bfloat16)
    csum = jax.lax.dot_general(
        ltri, mask.astype(jnp.bfloat16), (((1,), (0,)), ((), ())),
        preferred_element_type=jnp.float32)

    carry = cnt_acc[...]

    pair = jnp.concatenate([exps, (csum + carry) * mask], axis=1)
    ones_col = jnp.ones((NUM_EXPERTS, 1), jnp.float32)
    zeros_col = jnp.zeros((NUM_EXPERTS, 1), jnp.float32)
    sel = jnp.concatenate(
        [jnp.concatenate([ones_col, zeros_col], axis=1),
         jnp.concatenate([zeros_col, ones_col], axis=1)], axis=0)
    red = jax.lax.dot_general(
        pair, sel, (((1,), (0,)), ((), ())),
        preferred_element_type=jnp.float32)
    denom = red[:, 0:1]
    loc = red[:, 1:2]
    gate = 1.0 / denom

    me_part = jnp.sum(exps * gate, axis=0, keepdims=True)
    ce_part = jnp.sum(mask, axis=0, keepdims=True)

    idx_ref[...] = idx_f.astype(jnp.int32)
    loc_ref[...] = loc.astype(jnp.int32)
    gate_ref[...] = gate
    me_acc[...] += me_part
    cnt_acc[...] += ce_part

    @pl.when(i == NUM_BLOCKS - 1)
    def _fin():
        laux_ref[0, 0] = (jnp.sum(me_acc[...] * cnt_acc[...])
                          * (NUM_EXPERTS / (NUM_TOKENS * NUM_TOKENS)))


def kernel(input, W):
    num_tokens, num_experts = NUM_TOKENS, NUM_EXPERTS
    capacity = int((num_tokens + num_experts - 1) // num_experts)

    idx2, loc2, gate2, laux = pl.pallas_call(
        _gate_body,
        grid=(NUM_BLOCKS,),
        in_specs=[
            pl.BlockSpec((BLOCK_T, MODEL_DIM), lambda i: (i, 0)),
            pl.BlockSpec((NUM_EXPERTS, MODEL_DIM), lambda i: (0, 0)),
        ],
        out_specs=[
            pl.BlockSpec((BLOCK_T, 1), lambda i: (i, 0)),
            pl.BlockSpec((BLOCK_T, 1), lambda i: (i, 0)),
            pl.BlockSpec((BLOCK_T, 1), lambda i: (i, 0)),
            pl.BlockSpec(memory_space=pltpu.SMEM),
        ],
        out_shape=[
            jax.ShapeDtypeStruct((NUM_TOKENS, 1), jnp.int32),
            jax.ShapeDtypeStruct((NUM_TOKENS, 1), jnp.int32),
            jax.ShapeDtypeStruct((NUM_TOKENS, 1), jnp.float32),
            jax.ShapeDtypeStruct((1, 1), jnp.float32),
        ],
        scratch_shapes=[
            pltpu.VMEM((1, NUM_EXPERTS), jnp.float32),
            pltpu.VMEM((1, NUM_EXPERTS), jnp.float32),
        ],
    )(input, W)

    return (laux[0, 0], idx2[:, 0], capacity, loc2[:, 0], gate2[:, 0],
            num_experts)


# speedup vs baseline: 1.0582x; 1.0582x over previous
import jax
import jax.numpy as jnp
from jax.experimental import pallas as pl
from jax.experimental.pallas import tpu as pltpu

NUM_TOKENS = 32768
MODEL_DIM = 1024
NUM_EXPERTS = 64
BLOCK_T = 1024
NUM_BLOCKS = NUM_TOKENS // BLOCK_T


def _gate_body(x_ref, w_ref, ltri_ref, idx_ref, loc_ref, gate_ref, laux_ref,
               me_acc, cnt_acc):
    i = pl.program_id(0)

    @pl.when(i == 0)
    def _init():
        me_acc[...] = jnp.zeros_like(me_acc)
        cnt_acc[...] = jnp.zeros_like(cnt_acc)

    x = x_ref[...]
    w = w_ref[...]
    logits = jax.lax.dot_general(
        x, w, (((1,), (1,)), ((), ())),
        preferred_element_type=jnp.float32)

    rowmax = jnp.max(logits, axis=1, keepdims=True)
    eidx_f = jax.lax.broadcasted_iota(
        jnp.int32, (BLOCK_T, NUM_EXPERTS), 1).astype(jnp.float32)
    is_max = logits == rowmax
    idx_f = jnp.min(jnp.where(is_max, eidx_f, float(NUM_EXPERTS)),
                    axis=1, keepdims=True)

    exps = jnp.exp(logits - rowmax)
    mask = (eidx_f == idx_f).astype(jnp.float32)

    csum = jax.lax.dot_general(
        ltri_ref[...], mask.astype(jnp.bfloat16), (((1,), (0,)), ((), ())),
        preferred_element_type=jnp.float32)

    carry = cnt_acc[...]

    pair = jnp.concatenate([exps, (csum + carry) * mask], axis=1)
    ones_col = jnp.ones((NUM_EXPERTS, 1), jnp.float32)
    zeros_col = jnp.zeros((NUM_EXPERTS, 1), jnp.float32)
    sel = jnp.concatenate(
        [jnp.concatenate([ones_col, zeros_col], axis=1),
         jnp.concatenate([zeros_col, ones_col], axis=1)], axis=0)
    red = jax.lax.dot_general(
        pair, sel, (((1,), (0,)), ((), ())),
        preferred_element_type=jnp.float32)
    denom = red[:, 0:1]
    loc = red[:, 1:2]
    gate = 1.0 / denom

    me_part = jnp.sum(exps * gate, axis=0, keepdims=True)
    ce_part = jnp.sum(mask, axis=0, keepdims=True)

    idx_ref[...] = idx_f.astype(jnp.int32)
    loc_ref[...] = loc.astype(jnp.int32)
    gate_ref[...] = gate
    me_acc[...] += me_part
    cnt_acc[...] += ce_part

    @pl.when(i == NUM_BLOCKS - 1)
    def _fin():
        laux_ref[0, 0] = (jnp.sum(me_acc[...] * cnt_acc[...])
                          * (NUM_EXPERTS / (NUM_TOKENS * NUM_TOKENS)))


def kernel(input, W):
    num_tokens, num_experts = NUM_TOKENS, NUM_EXPERTS
    capacity = int((num_tokens + num_experts - 1) // num_experts)

    pallas_fn = pl.pallas_call(
        _gate_body,
        grid=(NUM_BLOCKS,),
        in_specs=[
            pl.BlockSpec((BLOCK_T, MODEL_DIM), lambda i: (i, 0)),
            pl.BlockSpec((NUM_EXPERTS, MODEL_DIM), lambda i: (0, 0)),
            pl.BlockSpec((BLOCK_T, BLOCK_T), lambda i: (0, 0)),
        ],
        out_specs=[
            pl.BlockSpec((BLOCK_T, 1), lambda i: (i, 0)),
            pl.BlockSpec((BLOCK_T, 1), lambda i: (i, 0)),
            pl.BlockSpec((BLOCK_T, 1), lambda i: (i, 0)),
            pl.BlockSpec(memory_space=pltpu.SMEM),
        ],
        out_shape=[
            jax.ShapeDtypeStruct((NUM_TOKENS, 1), jnp.int32),
            jax.ShapeDtypeStruct((NUM_TOKENS, 1), jnp.int32),
            jax.ShapeDtypeStruct((NUM_TOKENS, 1), jnp.float32),
            jax.ShapeDtypeStruct((1, 1), jnp.float32),
        ],
        scratch_shapes=[
            pltpu.VMEM((1, NUM_EXPERTS), jnp.float32),
            pltpu.VMEM((1, NUM_EXPERTS), jnp.float32),
        ],
    )
    r = jax.lax.broadcasted_iota(jnp.int32, (BLOCK_T, BLOCK_T), 0)
    c = jax.lax.broadcasted_iota(jnp.int32, (BLOCK_T, BLOCK_T), 1)
    ltri = (c < r).astype(jnp.bfloat16)
    idx2, loc2, gate2, laux = pallas_fn(input, W, ltri)

    return (laux[0, 0], idx2[:, 0], capacity, loc2[:, 0], gate2[:, 0],
            num_experts)


# speedup vs baseline: 1.0683x; 1.0095x over previous
import jax
import jax.numpy as jnp
from jax.experimental import pallas as pl
from jax.experimental.pallas import tpu as pltpu

NUM_TOKENS = 32768
MODEL_DIM = 1024
NUM_EXPERTS = 64
BLOCK_T = 1024
NUM_BLOCKS = NUM_TOKENS // BLOCK_T
NUM_STEPS = NUM_BLOCKS // 2


def _half(x, w, ltri, carry):
    logits = jax.lax.dot_general(
        x, w, (((1,), (1,)), ((), ())),
        preferred_element_type=jnp.float32)

    rowmax = jnp.max(logits, axis=1, keepdims=True)
    eidx_f = jax.lax.broadcasted_iota(
        jnp.int32, (BLOCK_T, NUM_EXPERTS), 1).astype(jnp.float32)
    is_max = logits == rowmax
    idx_f = jnp.min(jnp.where(is_max, eidx_f, float(NUM_EXPERTS)),
                    axis=1, keepdims=True)

    exps = jnp.exp(logits - rowmax)
    mask = (eidx_f == idx_f).astype(jnp.float32)

    csum = jax.lax.dot_general(
        ltri, mask.astype(jnp.bfloat16), (((1,), (0,)), ((), ())),
        preferred_element_type=jnp.float32)

    pair = jnp.concatenate([exps, (csum + carry) * mask], axis=1)
    ones_col = jnp.ones((NUM_EXPERTS, 1), jnp.float32)
    zeros_col = jnp.zeros((NUM_EXPERTS, 1), jnp.float32)
    sel = jnp.concatenate(
        [jnp.concatenate([ones_col, zeros_col], axis=1),
         jnp.concatenate([zeros_col, ones_col], axis=1)], axis=0)
    red = jax.lax.dot_general(
        pair, sel, (((1,), (0,)), ((), ())),
        preferred_element_type=jnp.float32)
    denom = red[:, 0:1]
    loc = red[:, 1:2]
    gate = 1.0 / denom

    me_part = jnp.sum(exps * gate, axis=0, keepdims=True)
    ce_part = jnp.sum(mask, axis=0, keepdims=True)
    return idx_f, loc, gate, me_part, ce_part


def _gate_body(xa_ref, xb_ref, w_ref, ltri_ref,
               idxa_ref, idxb_ref, loca_ref, locb_ref, gatea_ref, gateb_ref,
               laux_ref, me_acc, cnt_acc):
    i = pl.program_id(0)

    @pl.when(i == 0)
    def _init():
        me_acc[...] = jnp.zeros_like(me_acc)
        cnt_acc[...] = jnp.zeros_like(cnt_acc)

    w = w_ref[...]
    ltri = ltri_ref[...]

    idx_a, loc_a, gate_a, me_a, ce_a = _half(
        xa_ref[...], w, ltri, cnt_acc[...])
    idxa_ref[...] = idx_a.astype(jnp.int32)
    loca_ref[...] = loc_a.astype(jnp.int32)
    gatea_ref[...] = gate_a

    idx_b, loc_b, gate_b, me_b, ce_b = _half(
        xb_ref[...], w, ltri, cnt_acc[...] + ce_a)
    idxb_ref[...] = idx_b.astype(jnp.int32)
    locb_ref[...] = loc_b.astype(jnp.int32)
    gateb_ref[...] = gate_b

    me_acc[...] += me_a + me_b
    cnt_acc[...] += ce_a + ce_b

    @pl.when(i == NUM_STEPS - 1)
    def _fin():
        laux_ref[0, 0] = (jnp.sum(me_acc[...] * cnt_acc[...])
                          * (NUM_EXPERTS / (NUM_TOKENS * NUM_TOKENS)))


def kernel(input, W):
    num_tokens, num_experts = NUM_TOKENS, NUM_EXPERTS
    capacity = int((num_tokens + num_experts - 1) // num_experts)

    half_shape = jax.ShapeDtypeStruct((NUM_TOKENS // 2, 1), jnp.int32)
    half_shape_f = jax.ShapeDtypeStruct((NUM_TOKENS // 2, 1), jnp.float32)
    out = pl.pallas_call(
        _gate_body,
        grid=(NUM_STEPS,),
        in_specs=[
            pl.BlockSpec((BLOCK_T, MODEL_DIM), lambda i: (2 * i, 0)),
            pl.BlockSpec((BLOCK_T, MODEL_DIM), lambda i: (2 * i + 1, 0)),
            pl.BlockSpec((NUM_EXPERTS, MODEL_DIM), lambda i: (0, 0)),
            pl.BlockSpec((BLOCK_T, BLOCK_T), lambda i: (0, 0)),
        ],
        out_specs=[
            pl.BlockSpec((BLOCK_T, 1), lambda i: (i, 0)),
            pl.BlockSpec((BLOCK_T, 1), lambda i: (i, 0)),
            pl.BlockSpec((BLOCK_T, 1), lambda i: (i, 0)),
            pl.BlockSpec((BLOCK_T, 1), lambda i: (i, 0)),
            pl.BlockSpec((BLOCK_T, 1), lambda i: (i, 0)),
            pl.BlockSpec((BLOCK_T, 1), lambda i: (i, 0)),
            pl.BlockSpec(memory_space=pltpu.SMEM),
        ],
        out_shape=[
            half_shape, half_shape, half_shape, half_shape,
            half_shape_f, half_shape_f,
            jax.ShapeDtypeStruct((1, 1), jnp.float32),
        ],
        scratch_shapes=[
            pltpu.VMEM((1, NUM_EXPERTS), jnp.float32),
            pltpu.VMEM((1, NUM_EXPERTS), jnp.float32),
        ],
    )
    r = jax.lax.broadcasted_iota(jnp.int32, (BLOCK_T, BLOCK_T), 0)
    c = jax.lax.broadcasted_iota(jnp.int32, (BLOCK_T, BLOCK_T), 1)
    ltri = (c < r).astype(jnp.bfloat16)
    idxa, idxb, loca, locb, gatea, gateb, laux = out(input, input, W, ltri)

    def interleave(a, b):
        s = jnp.stack([a.reshape(NUM_STEPS, BLOCK_T),
                       b.reshape(NUM_STEPS, BLOCK_T)], axis=1)
        return s.reshape(NUM_TOKENS)

    return (laux[0, 0], interleave(idxa, idxb), capacity,
            interleave(loca, locb), interleave(gatea, gateb), num_experts)


# speedup vs baseline: 1.9768x; 1.8503x over previous
import jax
import jax.numpy as jnp
from jax.experimental import pallas as pl
from jax.experimental.pallas import tpu as pltpu

NUM_TOKENS = 32768
MODEL_DIM = 1024
NUM_EXPERTS = 64
BLOCK_T = 1024
NUM_BLOCKS = NUM_TOKENS // BLOCK_T


def _gate_body(x_ref, w_ref, utri_ref, eidx_ref,
               idx_ref, loc_ref, gate_ref, laux_ref, me_acc, cnt_acc):
    i = pl.program_id(0)

    @pl.when(i == 0)
    def _init():
        me_acc[...] = jnp.zeros_like(me_acc)
        cnt_acc[...] = jnp.zeros_like(cnt_acc)

    E, B = NUM_EXPERTS, BLOCK_T
    lg = jax.lax.dot_general(
        w_ref[...], x_ref[...], (((1,), (1,)), ((), ())),
        preferred_element_type=jnp.float32)

    eidx_f = eidx_ref[...]
    rowmax = jnp.max(lg, axis=0, keepdims=True)
    is_max = lg == rowmax
    idx_f = jnp.min(jnp.where(is_max, eidx_f, float(E)),
                    axis=0, keepdims=True)

    exps = jnp.exp(lg - rowmax)
    denom = jnp.sum(exps, axis=0, keepdims=True)
    gate = 1.0 / denom
    mask = (eidx_f == idx_f).astype(jnp.float32)

    csum = jax.lax.dot_general(
        mask.astype(jnp.bfloat16), utri_ref[...], (((1,), (0,)), ((), ())),
        preferred_element_type=jnp.float32)
    loc_local = jnp.sum(csum * mask, axis=0, keepdims=True)
    loc_carry = jax.lax.dot_general(
        cnt_acc[...], mask, (((1,), (0,)), ((), ())),
        preferred_element_type=jnp.float32)

    ones_row = jnp.ones((1, B), jnp.float32)
    me_part = jax.lax.dot_general(
        gate, exps, (((1,), (1,)), ((), ())),
        preferred_element_type=jnp.float32)
    ce_part = jax.lax.dot_general(
        ones_row, mask, (((1,), (1,)), ((), ())),
        preferred_element_type=jnp.float32)

    idx_ref[...] = idx_f.astype(jnp.int32).reshape(1, 1, B)
    loc_ref[...] = (loc_local + loc_carry).astype(jnp.int32).reshape(1, 1, B)
    gate_ref[...] = gate.reshape(1, 1, B)
    me_acc[...] += me_part
    cnt_acc[...] += ce_part

    @pl.when(i == NUM_BLOCKS - 1)
    def _fin():
        laux_ref[0, 0] = (jnp.sum(me_acc[...] * cnt_acc[...])
                          * (NUM_EXPERTS / (NUM_TOKENS * NUM_TOKENS)))


def kernel(input, W):
    num_tokens, num_experts = NUM_TOKENS, NUM_EXPERTS
    capacity = int((num_tokens + num_experts - 1) // num_experts)
    B = BLOCK_T

    row_i = jax.ShapeDtypeStruct((NUM_BLOCKS, 1, B), jnp.int32)
    row_f = jax.ShapeDtypeStruct((NUM_BLOCKS, 1, B), jnp.float32)
    pallas_fn = pl.pallas_call(
        _gate_body,
        grid=(NUM_BLOCKS,),
        in_specs=[
            pl.BlockSpec((B, MODEL_DIM), lambda i: (i, 0)),
            pl.BlockSpec((NUM_EXPERTS, MODEL_DIM), lambda i: (0, 0)),
            pl.BlockSpec((B, B), lambda i: (0, 0)),
            pl.BlockSpec((NUM_EXPERTS, B), lambda i: (0, 0)),
        ],
        out_specs=[
            pl.BlockSpec((1, 1, B), lambda i: (i, 0, 0)),
            pl.BlockSpec((1, 1, B), lambda i: (i, 0, 0)),
            pl.BlockSpec((1, 1, B), lambda i: (i, 0, 0)),
            pl.BlockSpec(memory_space=pltpu.SMEM),
        ],
        out_shape=[
            row_i, row_i, row_f,
            jax.ShapeDtypeStruct((1, 1), jnp.float32),
        ],
        scratch_shapes=[
            pltpu.VMEM((1, NUM_EXPERTS), jnp.float32),
            pltpu.VMEM((1, NUM_EXPERTS), jnp.float32),
        ],
    )

    s = jax.lax.broadcasted_iota(jnp.int32, (B, B), 0)
    t = jax.lax.broadcasted_iota(jnp.int32, (B, B), 1)
    utri = (s < t).astype(jnp.bfloat16)
    eidx = jax.lax.broadcasted_iota(
        jnp.int32, (num_experts, B), 0).astype(jnp.float32)

    idx3, loc3, gate3, laux = pallas_fn(input, W, utri, eidx)
    return (laux[0, 0], idx3.reshape(num_tokens), capacity,
            loc3.reshape(num_tokens), gate3.reshape(num_tokens), num_experts)
